# Initial kernel scaffold; baseline (speedup 1.0000x reference)
import jax
import jax.numpy as jnp
from jax.experimental import pallas as pl


def kernel(voxels, subject_ids, Wd, bd, Wu, bu, We, be, g_enc, b_enc, Wb, bb, g_bb, b_bb, Wi, bi, Wt, bt):
    raise NotImplementedError("write your pallas kernel here")


# speedup vs baseline: 1.4288x; 1.4288x over previous
import jax
import jax.numpy as jnp
from jax.experimental import pallas as pl
from jax.experimental.pallas import tpu as pltpu

S = 4
IN = 4096
H = 2048
D = 4
BN = 128
IMG = 768
TXT = 768
B = 1024

BM_E = 256
BM_S = 128


def _gelu(x):
    return 0.5 * x * (1.0 + jax.lax.erf(x * 0.7071067811865476))


def _ln(x, g, b, eps=1e-5):
    mu = jnp.mean(x, axis=-1, keepdims=True)
    var = jnp.mean((x - mu) ** 2, axis=-1, keepdims=True)
    return (x - mu) * jax.lax.rsqrt(var + eps) * g + b


def _encoder_body(x_ref, sid_ref, wd_ref, bd_ref, wu_ref, bu_ref, we_ref,
                  be_ref, g_ref, b_ref, out_ref):
    s = pl.program_id(1)
    x = x_ref[...]
    xb = x.astype(jnp.bfloat16)
    d = jnp.dot(xb, wd_ref[0], preferred_element_type=jnp.float32) + bd_ref[0]
    d = _gelu(d)
    u = jnp.dot(d.astype(jnp.bfloat16), wu_ref[0],
                preferred_element_type=jnp.float32)
    h = x + u + bu_ref[0]
    e = jnp.dot(h.astype(jnp.bfloat16), we_ref[0],
                preferred_element_type=jnp.float32) + be_ref[0]
    e = _ln(e, g_ref[0], b_ref[0])
    e = _gelu(e)
    mask = sid_ref[...] == s

    @pl.when(s == 0)
    def _():
        out_ref[...] = jnp.where(mask, e, 0.0)

    @pl.when(s > 0)
    def _():
        out_ref[...] = jnp.where(mask, e, out_ref[...])


def _stack_body(x_ref, wb_ref, bb_ref, g_ref, b_ref, wi_ref, bi_ref, wt_ref,
                bt_ref, img_ref, txt_ref):
    x = x_ref[...]
    for i in range(D):
        y = jnp.dot(x.astype(jnp.bfloat16), wb_ref[i],
                    preferred_element_type=jnp.float32) + bb_ref[i]
        y = _ln(y, g_ref[i], b_ref[i])
        x = x + _gelu(y)
    xb = x.astype(jnp.bfloat16)
    img_ref[...] = jnp.dot(xb, wi_ref[...],
                           preferred_element_type=jnp.float32) + bi_ref[...]
    txt_ref[...] = jnp.dot(xb, wt_ref[...],
                           preferred_element_type=jnp.float32) + bt_ref[...]


def kernel(voxels, subject_ids, Wd, bd, Wu, bu, We, be, g_enc, b_enc, Wb, bb,
           g_bb, b_bb, Wi, bi, Wt, bt):
    f32 = jnp.float32
    bf16 = jnp.bfloat16
    sid = subject_ids.astype(jnp.int32).reshape(B, 1)

    nb_e = B // BM_E
    feats = pl.pallas_call(
        _encoder_body,
        grid=(nb_e, S),
        in_specs=[
            pl.BlockSpec((BM_E, IN), lambda b, s: (b, 0)),
            pl.BlockSpec((BM_E, 1), lambda b, s: (b, 0)),
            pl.BlockSpec((1, IN, BN), lambda b, s: (s, 0, 0)),
            pl.BlockSpec((1, 1, BN), lambda b, s: (s, 0, 0)),
            pl.BlockSpec((1, BN, IN), lambda b, s: (s, 0, 0)),
            pl.BlockSpec((1, 1, IN), lambda b, s: (s, 0, 0)),
            pl.BlockSpec((1, IN, H), lambda b, s: (s, 0, 0)),
            pl.BlockSpec((1, 1, H), lambda b, s: (s, 0, 0)),
            pl.BlockSpec((1, 1, H), lambda b, s: (s, 0, 0)),
            pl.BlockSpec((1, 1, H), lambda b, s: (s, 0, 0)),
        ],
        out_specs=pl.BlockSpec((BM_E, H), lambda b, s: (b, 0)),
        out_shape=jax.ShapeDtypeStruct((B, H), f32),
        compiler_params=pltpu.CompilerParams(
            dimension_semantics=("arbitrary", "arbitrary")),
    )(
        voxels,
        sid,
        Wd.astype(bf16),
        bd.reshape(S, 1, BN),
        Wu.astype(bf16),
        bu.reshape(S, 1, IN),
        We.astype(bf16),
        be.reshape(S, 1, H),
        g_enc.reshape(S, 1, H),
        b_enc.reshape(S, 1, H),
    )

    nb_s = B // BM_S
    img, txt = pl.pallas_call(
        _stack_body,
        grid=(nb_s,),
        in_specs=[
            pl.BlockSpec((BM_S, H), lambda b: (b, 0)),
            pl.BlockSpec((D, H, H), lambda b: (0, 0, 0)),
            pl.BlockSpec((D, 1, H), lambda b: (0, 0, 0)),
            pl.BlockSpec((D, 1, H), lambda b: (0, 0, 0)),
            pl.BlockSpec((D, 1, H), lambda b: (0, 0, 0)),
            pl.BlockSpec((H, IMG), lambda b: (0, 0)),
            pl.BlockSpec((1, IMG), lambda b: (0, 0)),
            pl.BlockSpec((H, TXT), lambda b: (0, 0)),
            pl.BlockSpec((1, TXT), lambda b: (0, 0)),
        ],
        out_specs=[
            pl.BlockSpec((BM_S, IMG), lambda b: (b, 0)),
            pl.BlockSpec((BM_S, TXT), lambda b: (b, 0)),
        ],
        out_shape=[
            jax.ShapeDtypeStruct((B, IMG), f32),
            jax.ShapeDtypeStruct((B, TXT), f32),
        ],
        compiler_params=pltpu.CompilerParams(
            dimension_semantics=("arbitrary",)),
    )(
        feats,
        Wb.astype(bf16),
        bb.reshape(D, 1, H),
        g_bb.reshape(D, 1, H),
        b_bb.reshape(D, 1, H),
        Wi.astype(bf16),
        bi.reshape(1, IMG),
        Wt.astype(bf16),
        bt.reshape(1, TXT),
    )
    return img, txt
